# baseline (speedup 1.0000x reference)
import functools

import jax
import jax.numpy as jnp
from jax import lax
from jax.experimental import pallas as pl
from jax.experimental.pallas import tpu as pltpu

_NUM = 28
_IN_F = _NUM * _NUM
_HID = _NUM * 3
_OUT_F = 10
_BN_EPS = 1e-5

_CONTRACT_LAST = (((1,), (1,)), ((), ()))


def _round_up(n, m):
    return ((n + m - 1) // m) * m


def _fused_kernel(x_ref, w1_ref, w2_ref, g_ref, b_ref, out_ref,
                  x1_ref, w1s_ref, w2s_ref, sum_ref, sumsq_ref, *,
                  inv_b, tb):
    p = pl.program_id(0)
    i = pl.program_id(1)

    @pl.when((p == 0) & (i == 0))
    def _init():
        w1s_ref[...] = jnp.sign(w1_ref[...]).astype(jnp.bfloat16)
        w2s_ref[...] = jnp.sign(w2_ref[...]).astype(jnp.bfloat16)
        sum_ref[...] = jnp.zeros_like(sum_ref)
        sumsq_ref[...] = jnp.zeros_like(sumsq_ref)

    @pl.when(p == 0)
    def _fc1_stats():
        x1 = lax.dot_general(x_ref[...], w1s_ref[...], _CONTRACT_LAST,
                             preferred_element_type=jnp.float32)
        x1_ref[pl.ds(i * tb, tb), :] = x1
        sum_ref[...] += jnp.sum(x1, axis=0, keepdims=True)
        sumsq_ref[...] += jnp.sum(x1 * x1, axis=0, keepdims=True)

    @pl.when(p == 1)
    def _bn_fc2():
        mean = sum_ref[...] * inv_b
        var = sumsq_ref[...] * inv_b - mean * mean
        scale = lax.rsqrt(var + _BN_EPS) * g_ref[...]
        x1 = x1_ref[pl.ds(i * tb, tb), :]
        xn = (x1 - mean) * scale + b_ref[...]
        xb = jnp.sign(xn).astype(jnp.bfloat16)
        out_ref[...] = lax.dot_general(xb, w2s_ref[...], _CONTRACT_LAST,
                                       preferred_element_type=jnp.float32)


def kernel(x_nchw, w1, w2, gamma, beta):
    B = x_nchw.shape[0]

    x2d = x_nchw.reshape(B, _IN_F).astype(jnp.bfloat16)
    TB = 4096
    B_pad = _round_up(B, TB)
    if B_pad != B:
        x2d = jnp.pad(x2d, ((0, B_pad - B), (0, 0)))
    nt = B_pad // TB

    out = pl.pallas_call(
        functools.partial(_fused_kernel, inv_b=1.0 / B, tb=TB),
        out_shape=jax.ShapeDtypeStruct((B_pad, _OUT_F), jnp.float32),
        grid=(2, nt),
        in_specs=[
            pl.BlockSpec((TB, _IN_F),
                         lambda p, i: ((1 - p) * i + p * (nt - 1), 0)),
            pl.BlockSpec((_HID, _IN_F), lambda p, i: (0, 0)),
            pl.BlockSpec((_OUT_F, _HID), lambda p, i: (0, 0)),
            pl.BlockSpec((1, _HID), lambda p, i: (0, 0)),
            pl.BlockSpec((1, _HID), lambda p, i: (0, 0)),
        ],
        out_specs=pl.BlockSpec((TB, _OUT_F), lambda p, i: (p * i, 0)),
        scratch_shapes=[
            pltpu.VMEM((B_pad, _HID), jnp.float32),
            pltpu.VMEM((_HID, _IN_F), jnp.bfloat16),
            pltpu.VMEM((_OUT_F, _HID), jnp.bfloat16),
            pltpu.VMEM((1, _HID), jnp.float32),
            pltpu.VMEM((1, _HID), jnp.float32),
        ],
        compiler_params=pltpu.CompilerParams(
            dimension_semantics=("arbitrary", "arbitrary")),
        cost_estimate=pl.CostEstimate(
            flops=2 * B_pad * _IN_F * _HID + 2 * B_pad * _HID * _OUT_F,
            transcendentals=_HID,
            bytes_accessed=2 * B_pad * _IN_F + 4 * _IN_F * _HID
                           + 4 * B_pad * _OUT_F),
        name="gfnet_fused",
    )(x2d, w1.astype(jnp.float32), w2.astype(jnp.float32),
      gamma.astype(jnp.float32).reshape(1, _HID),
      beta.astype(jnp.float32).reshape(1, _HID))

    return out[:B]
